# baseline (speedup 1.0000x reference)
import jax
import jax.numpy as jnp
from jax.experimental import pallas as pl

B = 128
S = 32768
BLK = 4096


def _body(idx_ref, logits_ref, mask_ref, out_l_ref, out_m_ref):
    j = pl.program_id(0)
    cols = jax.lax.broadcasted_iota(jnp.int32, (B, BLK), 1) + j * BLK
    hot = cols == idx_ref[...]
    m = mask_ref[...] | hot
    out_m_ref[...] = m
    out_l_ref[...] = jnp.where(m, -jnp.inf, logits_ref[...])


def kernel(logits, mask, idxs):
    idxs2 = idxs.astype(jnp.int32).reshape(B, 1)
    out_l, out_m = pl.pallas_call(
        _body,
        grid=(S // BLK,),
        in_specs=[
            pl.BlockSpec((B, 1), lambda j: (0, 0)),
            pl.BlockSpec((B, BLK), lambda j: (0, j)),
            pl.BlockSpec((B, BLK), lambda j: (0, j)),
        ],
        out_specs=[
            pl.BlockSpec((B, BLK), lambda j: (0, j)),
            pl.BlockSpec((B, BLK), lambda j: (0, j)),
        ],
        out_shape=[
            jax.ShapeDtypeStruct((B, S), jnp.float32),
            jax.ShapeDtypeStruct((B, S), jnp.bool_),
        ],
    )(idxs2, logits, mask)
    return out_l, out_m
